# Initial kernel scaffold; baseline (speedup 1.0000x reference)
import jax
import jax.numpy as jnp
from jax.experimental import pallas as pl


def kernel(points, point_features, boxes3d):
    raise NotImplementedError("write your pallas kernel here")


# speedup vs baseline: 14.2978x; 14.2978x over previous
import functools

import jax
import jax.numpy as jnp
from jax import lax
from jax.experimental import pallas as pl
from jax.experimental.pallas import tpu as pltpu
from jax.experimental.pallas import tpu_sc as plsc

_NUM_SAMPLED = 512
_EXTRA = 1.0
_L = 16


def _sc_pool(pts_t, bparams, ftab, *, B, N, M, C):
    NC, NS = 2, 16
    NW = NC * NS
    BOXES = B * M
    BPW = BOXES // NW
    NP = N + 8
    K = _NUM_SAMPLED
    GCH = 128
    NCH = K // GCH

    mesh = plsc.VectorSubcoreMesh(
        core_axis_name="c", subcore_axis_name="s",
        num_cores=NC, num_subcores=NS)

    @functools.partial(
        pl.kernel,
        out_type=(
            jax.ShapeDtypeStruct((BOXES * K, C), jnp.float32),
            jax.ShapeDtypeStruct((BOXES * K,), jnp.float32),
            jax.ShapeDtypeStruct((BOXES * K,), jnp.float32),
            jax.ShapeDtypeStruct((BOXES * K,), jnp.float32),
            jax.ShapeDtypeStruct((BOXES,), jnp.int32),
            jax.ShapeDtypeStruct((BOXES * K,), jnp.int32),
        ),
        mesh=mesh,
        compiler_params=pltpu.CompilerParams(needs_layout_passes=False),
        scratch_types=[
            pltpu.VMEM((N,), jnp.float32),
            pltpu.VMEM((N,), jnp.float32),
            pltpu.VMEM((N,), jnp.float32),
            pltpu.VMEM((BPW, _L), jnp.float32),
            pltpu.VMEM((K + _L,), jnp.int32),
            pltpu.VMEM((NCH, GCH), jnp.int32),
            pltpu.VMEM((K,), jnp.int32),
            pltpu.VMEM((GCH, C), jnp.float32),
            pltpu.VMEM((K,), jnp.float32),
            pltpu.VMEM((K,), jnp.float32),
            pltpu.VMEM((K,), jnp.float32),
            pltpu.VMEM((BPW,), jnp.int32),
            pltpu.SemaphoreType.DMA,
        ],
    )
    def pool_kernel(pts_hbm, bp_hbm, ftab_hbm,
                    feat_hbm, x_hbm, y_hbm, z_hbm, flag_hbm, idx_hbm,
                    xs, ys, zs, bp, buf, gidx, oidx, fbuf,
                    xb, yb, zb, flags, sem):
        wid = lax.axis_index("s") * NC + lax.axis_index("c")
        base_box = wid * BPW
        batch = base_box // M
        pltpu.sync_copy(pts_hbm.at[batch * 3 + 0], xs)
        pltpu.sync_copy(pts_hbm.at[batch * 3 + 1], ys)
        pltpu.sync_copy(pts_hbm.at[batch * 3 + 2], zs)
        pltpu.sync_copy(bp_hbm.at[pl.ds(base_box, BPW)], bp)
        boff = batch * NP
        zrow = boff + N

        def box_body(j, flags_vec):
            pv = bp[j]
            cx = pv[0]
            cy = pv[1]
            cz = pv[2]
            hx = pv[3]
            hy = pv[4]
            hz = pv[5]
            ca = pv[6]
            sa = pv[7]

            def step(i, cnt):
                off = i * _L
                px = xs[pl.ds(off, _L)]
                py = ys[pl.ds(off, _L)]
                pz = zs[pl.ds(off, _L)]
                sx = px - cx
                sy = py - cy
                lx = sx * ca - sy * sa
                ly = sx * sa + sy * ca
                m = ((jnp.abs(pz - cz) <= hz)
                     & (lx > -hx) & (lx < hx)
                     & (ly > -hy) & (ly < hy))
                mi = jnp.where(m, jnp.int32(1), jnp.int32(0))
                incl = plsc.cumsum(mi)

                @pl.when(cnt < K)
                def _():
                    plsc.store_scatter(
                        buf, [cnt + incl - 1],
                        off + lax.iota(jnp.int32, _L), mask=m)

                return cnt + incl[_L - 1]

            cnt = lax.fori_loop(0, N // _L, step, jnp.int32(0))

            nonempty = cnt > 0
            safe = lax.broadcast(jnp.maximum(cnt, 1), (_L,))
            fzero = jnp.float32(0.0)
            for c in range(K // _L):
                kv = lax.iota(jnp.int32, _L) + (c * _L)
                p = lax.rem(kv, safe)
                g = plsc.load_gather(buf, [p])
                gs = jnp.where(nonempty, g, 0)
                oidx[pl.ds(c * _L, _L)] = gs
                gidx[c // (GCH // _L), pl.ds((c % (GCH // _L)) * _L, _L)] = (
                    jnp.where(nonempty, g + boff, zrow))
                xb[pl.ds(c * _L, _L)] = jnp.where(
                    nonempty, plsc.load_gather(xs, [gs]), fzero)
                yb[pl.ds(c * _L, _L)] = jnp.where(
                    nonempty, plsc.load_gather(ys, [gs]), fzero)
                zb[pl.ds(c * _L, _L)] = jnp.where(
                    nonempty, plsc.load_gather(zs, [gs]), fzero)

            boxg = base_box + j
            for r in range(NCH):
                pltpu.async_copy(ftab_hbm.at[gidx.at[r]], fbuf, sem).wait()
                pltpu.sync_copy(
                    fbuf, feat_hbm.at[pl.ds((boxg * K + r * GCH), GCH)])
            pltpu.sync_copy(oidx, idx_hbm.at[pl.ds(boxg * K, K)])
            pltpu.sync_copy(xb, x_hbm.at[pl.ds(boxg * K, K)])
            pltpu.sync_copy(yb, y_hbm.at[pl.ds(boxg * K, K)])
            pltpu.sync_copy(zb, z_hbm.at[pl.ds(boxg * K, K)])

            empty = jnp.where(cnt == 0, jnp.int32(1), jnp.int32(0))
            return jnp.where(lax.iota(jnp.int32, _L) == j, empty, flags_vec)

        flags_vec = lax.fori_loop(0, BPW, box_body,
                                  jnp.zeros((_L,), jnp.int32))
        flags[...] = flags_vec
        pltpu.sync_copy(flags, flag_hbm.at[pl.ds(base_box, BPW)])

    return pool_kernel(pts_t, bparams, ftab)


def kernel(points, point_features, boxes3d):
    B, N, _ = points.shape
    M = boxes3d.shape[1]
    C = point_features.shape[2]
    K = _NUM_SAMPLED

    pts_t = jnp.transpose(points, (0, 2, 1)).reshape(B * 3, N)
    rz = boxes3d[..., 6]
    half = (boxes3d[..., 3:6] + 2.0 * _EXTRA) / 2.0
    zcol = jnp.zeros_like(rz)
    bparams = jnp.stack(
        [boxes3d[..., 0], boxes3d[..., 1], boxes3d[..., 2],
         half[..., 0], half[..., 1], half[..., 2],
         jnp.cos(-rz), jnp.sin(-rz)] + [zcol] * (_L - 8),
        axis=-1).reshape(B * M, _L)
    ftab = jnp.concatenate(
        [point_features, jnp.zeros((B, 8, C), jnp.float32)], axis=1
    ).reshape(B * (N + 8), C)

    feat, x, y, z, flags, idx = _sc_pool(
        pts_t, bparams, ftab, B=B, N=N, M=M, C=C)

    xyz = jnp.stack([x, y, z], axis=-1).reshape(B, M, K, 3)
    pooled = jnp.concatenate([xyz, feat.reshape(B, M, K, C)], axis=-1)
    return (pooled, flags.reshape(B, M), idx.reshape(B, M, K))
